# baseline (speedup 1.0000x reference)
import functools

import jax
import jax.numpy as jnp
from jax.experimental import pallas as pl
from jax.experimental.pallas import tpu as pltpu

MEM = 8192
BS = 256


def _tables_body(h_ref, Wk_ref, bk_ref, Wv_ref, bv_ref, kn_ref, v_ref):
    hbar = jnp.mean(h_ref[:], axis=0)
    k = jnp.dot(hbar, Wk_ref[:], preferred_element_type=jnp.float32)
    k = k + bk_ref[0]
    n = jnp.sqrt(jnp.sum(k * k, axis=-1, keepdims=True))
    kn_ref[:] = k / jnp.maximum(n, 1e-12)
    v = jnp.dot(hbar, Wv_ref[:], preferred_element_type=jnp.float32)
    v_ref[:] = v + bv_ref[0]


def _attn_body(h_blk_ref, Wq_ref, bq_ref, kn_ref, v_ref,
               Wor_ref, Woh_ref, bo_ref, out_ref):
    h = h_blk_ref[0]
    q = jnp.dot(h, Wq_ref[:], preferred_element_type=jnp.float32) + bq_ref[0]
    n = jnp.sqrt(jnp.sum(q * q, axis=-1, keepdims=True))
    qn = q / jnp.maximum(n, 1e-12)
    sim = jax.lax.dot_general(qn, kn_ref[:], (((1,), (1,)), ((), ())),
                              preferred_element_type=jnp.float32)
    m = jnp.max(sim, axis=-1, keepdims=True)
    e = jnp.exp(sim - m)
    attn = e / jnp.sum(e, axis=-1, keepdims=True)
    r = jnp.dot(attn, v_ref[:], preferred_element_type=jnp.float32)
    out = jnp.dot(r, Wor_ref[:], preferred_element_type=jnp.float32)
    out += jnp.dot(h, Woh_ref[:], preferred_element_type=jnp.float32)
    out_ref[0] = out + bo_ref[0]


@jax.jit
def kernel(hidden_states, Wq, bq, Wk, bk, Wv, bv, Wo, bo):
    B, S, H = hidden_states.shape
    K = Wq.shape[1]
    V = Wv.shape[1]
    assert MEM % S == 0 and B > 1
    nblk = S // BS

    Wor = Wo[:V]
    Woh = Wo[V:]

    kn, vals = pl.pallas_call(
        _tables_body,
        out_shape=[jax.ShapeDtypeStruct((S, K), jnp.float32),
                   jax.ShapeDtypeStruct((S, V), jnp.float32)],
    )(hidden_states, Wk, bk.reshape(1, K), Wv, bv.reshape(1, V))

    out = pl.pallas_call(
        _attn_body,
        grid=(B, nblk),
        in_specs=[
            pl.BlockSpec((1, BS, H), lambda b, i: (b, i, 0)),
            pl.BlockSpec((H, K), lambda b, i: (0, 0)),
            pl.BlockSpec((1, K), lambda b, i: (0, 0)),
            pl.BlockSpec((S, K), lambda b, i: (0, 0)),
            pl.BlockSpec((S, V), lambda b, i: (0, 0)),
            pl.BlockSpec((V, H), lambda b, i: (0, 0)),
            pl.BlockSpec((H, H), lambda b, i: (0, 0)),
            pl.BlockSpec((1, H), lambda b, i: (0, 0)),
        ],
        out_specs=pl.BlockSpec((1, BS, H), lambda b, i: (b, i, 0)),
        out_shape=jax.ShapeDtypeStruct((B, S, H), jnp.float32),
        compiler_params=pltpu.CompilerParams(
            dimension_semantics=("parallel", "parallel")),
    )(hidden_states, Wq, bq.reshape(1, K), kn, vals, Wor, Woh,
      bo.reshape(1, H))
    return out


# speedup vs baseline: 1.3067x; 1.3067x over previous
import functools

import jax
import jax.numpy as jnp
from jax.experimental import pallas as pl
from jax.experimental.pallas import tpu as pltpu

MEM = 8192
BS = 256


def _body(h_full_ref, h_blk_ref, Wq_ref, bq_ref, Wk_ref, bk_ref,
          Wv_ref, bv_ref, Wor_ref, Woh_ref, bo_ref,
          out_ref, kn_s, v_s):
    b = pl.program_id(0)
    i = pl.program_id(1)

    @pl.when((b == 0) & (i == 0))
    def _init():
        hbar = jnp.mean(h_full_ref[:], axis=0)
        k = jnp.dot(hbar, Wk_ref[:], preferred_element_type=jnp.float32)
        k = k + bk_ref[0]
        n = jnp.sqrt(jnp.sum(k * k, axis=-1, keepdims=True))
        kn_s[:] = k / jnp.maximum(n, 1e-12)
        v = jnp.dot(hbar, Wv_ref[:], preferred_element_type=jnp.float32)
        v_s[:] = v + bv_ref[0]

    h = h_blk_ref[0]
    q = jnp.dot(h, Wq_ref[:], preferred_element_type=jnp.float32) + bq_ref[0]
    n = jnp.sqrt(jnp.sum(q * q, axis=-1, keepdims=True))
    qn = q / jnp.maximum(n, 1e-12)
    sim = jax.lax.dot_general(qn, kn_s[:], (((1,), (1,)), ((), ())),
                              preferred_element_type=jnp.float32)
    e = jnp.exp(sim)
    denom = jnp.sum(e, axis=-1, keepdims=True)
    r = jnp.dot(e, v_s[:], preferred_element_type=jnp.float32) / denom
    out = jnp.dot(r, Wor_ref[:], preferred_element_type=jnp.float32)
    out += jnp.dot(h, Woh_ref[:], preferred_element_type=jnp.float32)
    out_ref[0] = out + bo_ref[0]


@jax.jit
def kernel(hidden_states, Wq, bq, Wk, bk, Wv, bv, Wo, bo):
    B, S, H = hidden_states.shape
    K = Wq.shape[1]
    V = Wv.shape[1]
    assert MEM % S == 0 and B > 1
    nblk = S // BS

    Wor = Wo[:V]
    Woh = Wo[V:]

    grid = (B, nblk)
    out = pl.pallas_call(
        _body,
        grid=grid,
        in_specs=[
            pl.BlockSpec((B, S, H), lambda b, i: (0, 0, 0)),
            pl.BlockSpec((1, BS, H), lambda b, i: (b, i, 0)),
            pl.BlockSpec((H, K), lambda b, i: (0, 0)),
            pl.BlockSpec((1, K), lambda b, i: (0, 0)),
            pl.BlockSpec((H, K), lambda b, i: (0, 0)),
            pl.BlockSpec((1, K), lambda b, i: (0, 0)),
            pl.BlockSpec((H, V), lambda b, i: (0, 0)),
            pl.BlockSpec((1, V), lambda b, i: (0, 0)),
            pl.BlockSpec((V, H), lambda b, i: (0, 0)),
            pl.BlockSpec((H, H), lambda b, i: (0, 0)),
            pl.BlockSpec((1, H), lambda b, i: (0, 0)),
        ],
        out_specs=pl.BlockSpec((1, BS, H), lambda b, i: (b, i, 0)),
        out_shape=jax.ShapeDtypeStruct((B, S, H), jnp.float32),
        scratch_shapes=[
            pltpu.VMEM((S, K), jnp.float32),
            pltpu.VMEM((S, V), jnp.float32),
        ],
    )(hidden_states, hidden_states, Wq, bq.reshape(1, K), Wk,
      bk.reshape(1, K), Wv, bv.reshape(1, V), Wor, Woh, bo.reshape(1, H))
    return out
